# baseline (speedup 1.0000x reference)
import functools

import jax
import jax.numpy as jnp
from jax import lax
from jax.experimental import pallas as pl
from jax.experimental.pallas import tpu as pltpu
from jax.experimental.pallas import tpu_sc as plsc

D = 768
LANES = 16
NC = 2
NS = 16
NW = NC * NS
CH = 32
NBUF = 3


def _embed_sc(x_flat, W, pos_embd, batch, seq_len):
    P = seq_len // NW
    rows_per_w = batch * P
    n_chunks = rows_per_w // CH
    per_batch = P // CH
    B = x_flat.shape[0]
    mesh = plsc.VectorSubcoreMesh(core_axis_name="c", subcore_axis_name="s")

    @functools.partial(
        pl.kernel,
        out_type=jax.ShapeDtypeStruct((B, D), jnp.float32),
        mesh=mesh,
        scratch_types=(
            [pltpu.VMEM((rows_per_w,), jnp.int32),
             pltpu.VMEM((P, D), jnp.float32)]
            + [pltpu.VMEM((CH, D), jnp.float32)] * NBUF
            + [pltpu.SemaphoreType.DMA] * (2 * NBUF + 1)
        ),
    )
    def k(x_hbm, w_hbm, pos_hbm, out_hbm, idx_v, pos_v, *rest):
        bufs = rest[:NBUF]
        gsems = rest[NBUF:2 * NBUF]
        ssems = rest[2 * NBUF:3 * NBUF]
        psem = rest[3 * NBUF]

        wid = lax.axis_index("s") * NC + lax.axis_index("c")
        p0 = wid * P
        idx_cps = [
            pltpu.async_copy(x_hbm.at[pl.ds(b * seq_len + p0, P)],
                             idx_v.at[pl.ds(b * P, P)], psem)
            for b in range(batch)
        ]
        for cp in idx_cps:
            cp.wait()
        pos_cp = pltpu.async_copy(pos_hbm.at[pl.ds(p0, P)], pos_v, psem)

        gathers = {}
        scatters = {}

        def do_add(buf, po):
            @plsc.parallel_loop(0, CH, unroll=4)
            def add_row(r):
                for j in range(D // LANES):
                    sl = pl.ds(j * LANES, LANES)
                    buf[r, sl] = buf[r, sl] + pos_v[po + r, sl]

        def issue_gather(c):
            gathers[c] = pltpu.async_copy(
                w_hbm.at[idx_v.at[pl.ds(c * CH, CH)]],
                bufs[c % NBUF], gsems[c % NBUF])

        def issue_scatter(c):
            b, kk = divmod(c, per_batch)
            row0 = b * seq_len + p0 + kk * CH
            scatters[c] = pltpu.async_copy(
                bufs[c % NBUF], out_hbm.at[pl.ds(row0, CH)], ssems[c % NBUF])

        issue_gather(0)
        for c in range(n_chunks):
            if c + 1 < n_chunks:
                if c - (NBUF - 1) >= 0:
                    scatters[c - (NBUF - 1)].wait()
                issue_gather(c + 1)
            gathers[c].wait()
            if c == 0:
                pos_cp.wait()
            do_add(bufs[c % NBUF], (c % per_batch) * CH)
            issue_scatter(c)
        for c in range(max(0, n_chunks - NBUF + 1), n_chunks):
            scatters[c].wait()

    return k(x_flat, W, pos_embd)


def kernel(x, W, pos_embd):
    batch, seq_len = x.shape
    x_flat = x.reshape(-1).astype(jnp.int32)
    out = _embed_sc(x_flat, W, pos_embd[:seq_len], batch, seq_len)
    return out.reshape(batch, seq_len, D)


# speedup vs baseline: 1.0545x; 1.0545x over previous
import functools

import jax
import jax.numpy as jnp
from jax import lax
from jax.experimental import pallas as pl
from jax.experimental.pallas import tpu as pltpu
from jax.experimental.pallas import tpu_sc as plsc

D = 768
LANES = 16
NC = 2
NS = 16
NW = NC * NS
CH = 32
NBUF = 3


def _embed_sc(x_flat, W, pos_embd, batch, seq_len):
    P = seq_len // NW
    rows_per_w = batch * P
    n_chunks = rows_per_w // CH
    per_batch = P // CH
    B = x_flat.shape[0]
    mesh = plsc.VectorSubcoreMesh(core_axis_name="c", subcore_axis_name="s")

    @functools.partial(
        pl.kernel,
        out_type=jax.ShapeDtypeStruct((B, D), jnp.float32),
        mesh=mesh,
        scratch_types=(
            [pltpu.VMEM((rows_per_w,), jnp.int32),
             pltpu.VMEM((P, D), jnp.float32)]
            + [pltpu.VMEM((CH, D), jnp.float32)] * NBUF
            + [pltpu.SemaphoreType.DMA] * (2 * NBUF + 1 + batch)
        ),
    )
    def k(x_hbm, w_hbm, pos_hbm, out_hbm, idx_v, pos_v, *rest):
        bufs = rest[:NBUF]
        gsems = rest[NBUF:2 * NBUF]
        ssems = rest[2 * NBUF:3 * NBUF]
        psem = rest[3 * NBUF]
        isems = rest[3 * NBUF + 1:]

        wid = lax.axis_index("s") * NC + lax.axis_index("c")
        p0 = wid * P
        idx_cps = [
            pltpu.async_copy(x_hbm.at[pl.ds(b * seq_len + p0, P)],
                             idx_v.at[pl.ds(b * P, P)], isems[b])
            for b in range(batch)
        ]
        idx_done = set()

        def need_idx(c):
            b = c // per_batch
            if b not in idx_done:
                idx_cps[b].wait()
                idx_done.add(b)

        gathers = {}
        scatters = {}

        def do_add(buf, po):
            @plsc.parallel_loop(0, CH, unroll=2)
            def add_row(r):
                for j in range(D // LANES):
                    sl = pl.ds(j * LANES, LANES)
                    buf[r, sl] = buf[r, sl] + pos_v[po + r, sl]

        def issue_gather(c):
            gathers[c] = pltpu.async_copy(
                w_hbm.at[idx_v.at[pl.ds(c * CH, CH)]],
                bufs[c % NBUF], gsems[c % NBUF])

        def issue_scatter(c):
            b, kk = divmod(c, per_batch)
            row0 = b * seq_len + p0 + kk * CH
            scatters[c] = pltpu.async_copy(
                bufs[c % NBUF], out_hbm.at[pl.ds(row0, CH)], ssems[c % NBUF])

        need_idx(0)
        issue_gather(0)
        pos_cp = pltpu.async_copy(pos_hbm.at[pl.ds(p0, P)], pos_v, psem)
        for c in range(n_chunks):
            if c + 1 < n_chunks:
                need_idx(c + 1)
                if c - (NBUF - 1) >= 0:
                    scatters[c - (NBUF - 1)].wait()
                issue_gather(c + 1)
            gathers[c].wait()
            if c == 0:
                pos_cp.wait()
            do_add(bufs[c % NBUF], (c % per_batch) * CH)
            issue_scatter(c)
        for c in range(max(0, n_chunks - NBUF + 1), n_chunks):
            scatters[c].wait()

    return k(x_flat, W, pos_embd)


def kernel(x, W, pos_embd):
    batch, seq_len = x.shape
    x_flat = x.reshape(-1).astype(jnp.int32)
    out = _embed_sc(x_flat, W, pos_embd[:seq_len], batch, seq_len)
    return out.reshape(batch, seq_len, D)


# speedup vs baseline: 1.0755x; 1.0199x over previous
import functools

import jax
import jax.numpy as jnp
from jax import lax
from jax.experimental import pallas as pl
from jax.experimental.pallas import tpu as pltpu
from jax.experimental.pallas import tpu_sc as plsc

D = 768
LANES = 16
NC = 2
NS = 16
NW = NC * NS
CH = 32
NBUF = 3


def _embed_sc(x_flat, W, pos_embd, batch, seq_len):
    P = seq_len // NW
    rows_per_w = batch * P
    n_chunks = rows_per_w // CH
    per_batch = P // CH
    B = x_flat.shape[0]
    mesh = plsc.VectorSubcoreMesh(core_axis_name="c", subcore_axis_name="s")

    @functools.partial(
        pl.kernel,
        out_type=jax.ShapeDtypeStruct((B, D), jnp.float32),
        mesh=mesh,
        scratch_types=(
            [pltpu.VMEM((rows_per_w,), jnp.int32),
             pltpu.VMEM((P, D), jnp.float32)]
            + [pltpu.VMEM((CH, D), jnp.float32)] * NBUF
            + [pltpu.SemaphoreType.DMA] * (2 * NBUF + 1 + batch)
        ),
    )
    def k(x_hbm, w_hbm, pos_hbm, out_hbm, idx_v, pos_v, *rest):
        bufs = rest[:NBUF]
        gsems = rest[NBUF:2 * NBUF]
        ssems = rest[2 * NBUF:3 * NBUF]
        psem = rest[3 * NBUF]
        isems = rest[3 * NBUF + 1:]

        wid = lax.axis_index("s") * NC + lax.axis_index("c")
        p0 = wid * P
        idx_cps = [
            pltpu.async_copy(x_hbm.at[pl.ds(b * seq_len + p0, P)],
                             idx_v.at[pl.ds(b * P, P)], isems[b])
            for b in range(batch)
        ]
        idx_done = set()

        def need_idx(c):
            b = c // per_batch
            if b not in idx_done:
                idx_cps[b].wait()
                idx_done.add(b)

        gathers = {}
        scatters = {}

        def do_add(buf, po):
            @plsc.parallel_loop(0, CH, unroll=2)
            def add_row(r):
                for j in range(D // LANES):
                    sl = pl.ds(j * LANES, LANES)
                    plsc.addupdate(buf.at[r, sl], pos_v[po + r, sl])

        def issue_gather(c):
            gathers[c] = pltpu.async_copy(
                w_hbm.at[idx_v.at[pl.ds(c * CH, CH)]],
                bufs[c % NBUF], gsems[c % NBUF])

        def issue_scatter(c):
            b, kk = divmod(c, per_batch)
            row0 = b * seq_len + p0 + kk * CH
            scatters[c] = pltpu.async_copy(
                bufs[c % NBUF], out_hbm.at[pl.ds(row0, CH)], ssems[c % NBUF])

        need_idx(0)
        issue_gather(0)
        pos_cp = pltpu.async_copy(pos_hbm.at[pl.ds(p0, P)], pos_v, psem)
        for c in range(n_chunks):
            if c + 1 < n_chunks:
                need_idx(c + 1)
                if c - (NBUF - 1) >= 0:
                    scatters[c - (NBUF - 1)].wait()
                issue_gather(c + 1)
            gathers[c].wait()
            if c == 0:
                pos_cp.wait()
            do_add(bufs[c % NBUF], (c % per_batch) * CH)
            issue_scatter(c)
        for c in range(max(0, n_chunks - NBUF + 1), n_chunks):
            scatters[c].wait()

    return k(x_flat, W, pos_embd)


def kernel(x, W, pos_embd):
    batch, seq_len = x.shape
    x_flat = x.reshape(-1).astype(jnp.int32)
    out = _embed_sc(x_flat, W, pos_embd[:seq_len], batch, seq_len)
    return out.reshape(batch, seq_len, D)
